# baseline (speedup 1.0000x reference)
import jax
import jax.numpy as jnp
from jax import lax
from jax.experimental import pallas as pl
from jax.experimental.pallas import tpu as pltpu
from jax.experimental.pallas import tpu_sc as plsc

_LANES = 16
_NC = 2
_NS = 16
_NW = _NC * _NS
_CHUNK = 80
_NCHUNKS = 10000
_ROWP = 64
_SCC = 25
_DEPTH = 5
_CHUNKS_C0 = 600
_CHUNKS_C1 = 25


def _sc_body(gt_hbm, est_hbm, gconn_hbm, econn_hbm, out_hbm,
             ixg, ixe,
             rg0, re0, rg1, re1, rg2, re2, rg3, re3, rg4, re4,
             accv, semi, sem0, sem1, sem2, sem3, sem4):
    cid = lax.axis_index("c")
    sid = lax.axis_index("s")
    wid = sid * _NC + cid
    chunk0 = jnp.where(cid == 0, sid * _CHUNKS_C0,
                       _NS * _CHUNKS_C0 + sid * _CHUNKS_C1)
    nsc = jnp.where(cid == 0, _CHUNKS_C0 // _SCC, _CHUNKS_C1 // _SCC)

    bufs = ((rg0, re0, sem0), (rg1, re1, sem1), (rg2, re2, sem2),
            (rg3, re3, sem3), (rg4, re4, sem4))

    def fire(c, b):
        rg_, re_, sem = bufs[b]
        sl = pl.ds(c * 2 * _CHUNK, 2 * _CHUNK)
        pltpu.async_copy(gt_hbm.at[ixg.at[sl]], rg_, sem)
        pltpu.async_copy(est_hbm.at[ixe.at[sl]], re_, sem)

    def wait(b):
        rg_, re_, sem = bufs[b]
        sl = pl.ds(0, 2 * _CHUNK)
        pltpu.make_async_copy(gt_hbm.at[ixg.at[sl]], rg_, sem).wait()
        pltpu.make_async_copy(est_hbm.at[ixe.at[sl]], re_, sem).wait()

    def compute(b, accs):
        rg_, re_, _ = bufs[b]

        def row2(rr, accs_in):
            outs = list(accs_in)
            for u in range(2):
                r = 2 * (rr * 2 + u)
                for h in range(2):
                    sl = pl.ds(h * 32, 32)
                    d = ((rg_[r, sl] - rg_[r + 1, sl])
                         - (re_[r, sl] - re_[r + 1, sl]))
                    lo, hi = plsc.unpack(d, format=plsc.PackFormat.INTERLEAVED)
                    j = 2 * h
                    outs[j] = outs[j] + lo * lo
                    outs[j + 1] = outs[j + 1] + hi * hi
            return tuple(outs)

        return lax.fori_loop(0, _CHUNK // 2, row2, accs)

    def superchunk(s, accs):
        base = pl.ds((chunk0 + s * _SCC) * 2 * _CHUNK, _SCC * 2 * _CHUNK)
        cps = [pltpu.async_copy(gconn_hbm.at[base], ixg, semi),
               pltpu.async_copy(econn_hbm.at[base], ixe, semi)]
        for cp in cps:
            cp.wait()
        for b in range(_DEPTH):
            fire(b, b)

        def block(j, accs_in):
            for u in range(_DEPTH):
                wait(u)
                accs_in = compute(u, accs_in)
                fire(_DEPTH * j + u + _DEPTH, u)
            return accs_in

        accs = lax.fori_loop(0, _SCC // _DEPTH - 1, block, accs)
        for u in range(_DEPTH):
            wait(u)
            accs = compute(u, accs)
        return accs

    zero = jnp.zeros((_LANES,), jnp.float32)
    accs = lax.fori_loop(0, nsc, superchunk, (zero, zero, zero, zero))
    accv[...] = (accs[0] + accs[1]) + (accs[2] + accs[3])
    pltpu.sync_copy(accv, out_hbm.at[wid])


def kernel(gt_vertices, est_vertices, gt_connections, est_connections):
    B, N, C3 = gt_vertices.shape
    row = C3 * B

    zpad = jnp.zeros((N, _ROWP - row), jnp.bfloat16)
    gtT = jnp.concatenate(
        [gt_vertices.reshape(B, N * C3).T.reshape(N, row)
         .astype(jnp.bfloat16), zpad], axis=1)
    estT = jnp.concatenate(
        [est_vertices.reshape(B, N * C3).T.reshape(N, row)
         .astype(jnp.bfloat16), zpad], axis=1)
    conn_g = gt_connections.astype(jnp.int32).reshape(-1)
    conn_e = est_connections.astype(jnp.int32).reshape(-1)

    idxbuf = pltpu.VMEM((_SCC * 2 * _CHUNK,), jnp.int32)
    rowbuf = pltpu.VMEM((2 * _CHUNK, _ROWP), jnp.bfloat16)
    run = pl.kernel(
        _sc_body,
        mesh=plsc.VectorSubcoreMesh(core_axis_name="c", subcore_axis_name="s"),
        compiler_params=pltpu.CompilerParams(use_tc_tiling_on_sc=False,
                                             needs_layout_passes=False),
        out_type=jax.ShapeDtypeStruct((_NW, _LANES), jnp.float32),
        scratch_types=(
            [idxbuf] * 2
            + [rowbuf] * (2 * _DEPTH)
            + [pltpu.VMEM((_LANES,), jnp.float32)]
            + [pltpu.SemaphoreType.DMA] * (1 + _DEPTH)
        ),
    )
    partials = run(gtT, estT, conn_g, conn_e)
    return jnp.sum(partials) / (B + 1e-08)


# speedup vs baseline: 2.8700x; 2.8700x over previous
import jax
import jax.numpy as jnp
from jax import lax
from jax.experimental import pallas as pl
from jax.experimental.pallas import tpu as pltpu
from jax.experimental.pallas import tpu_sc as plsc

_LANES = 16
_NC = 2
_NS = 16
_NW = _NC * _NS
_CHUNK = 128
_E_PAD = 819200
_NCHUNKS = _E_PAD // _CHUNK
_ROWP = 64
_SCC = 16
_DEPTH = 4
_CHUNKS_C0 = 384
_CHUNKS_C1 = 16


def _sc_body(gt_hbm, est_hbm, ig0_hbm, ig1_hbm, ie0_hbm, ie1_hbm, out_hbm,
             ix0, ix1, ix2, ix3,
             ra0, rb0, rc0, rd0, ra1, rb1, rc1, rd1,
             ra2, rb2, rc2, rd2, ra3, rb3, rc3, rd3,
             accv, semi, sem0, sem1, sem2, sem3):
    cid = lax.axis_index("c")
    sid = lax.axis_index("s")
    wid = sid * _NC + cid
    chunk0 = jnp.where(cid == 0, sid * _CHUNKS_C0,
                       _NS * _CHUNKS_C0 + sid * _CHUNKS_C1)
    nsc = jnp.where(cid == 0, _CHUNKS_C0 // _SCC, _CHUNKS_C1 // _SCC)

    bufs = ((ra0, rb0, rc0, rd0, sem0), (ra1, rb1, rc1, rd1, sem1),
            (ra2, rb2, rc2, rd2, sem2), (ra3, rb3, rc3, rd3, sem3))

    def fire(c, b):
        ra_, rb_, rc_, rd_, sem = bufs[b]
        sl = pl.ds(c * _CHUNK, _CHUNK)
        pltpu.async_copy(gt_hbm.at[ix0.at[sl]], ra_, sem)
        pltpu.async_copy(gt_hbm.at[ix1.at[sl]], rb_, sem)
        pltpu.async_copy(est_hbm.at[ix2.at[sl]], rc_, sem)
        pltpu.async_copy(est_hbm.at[ix3.at[sl]], rd_, sem)

    def wait(b):
        ra_, rb_, rc_, rd_, sem = bufs[b]
        sl = pl.ds(0, _CHUNK)
        pltpu.make_async_copy(gt_hbm.at[ix0.at[sl]], ra_, sem).wait()
        pltpu.make_async_copy(gt_hbm.at[ix1.at[sl]], rb_, sem).wait()
        pltpu.make_async_copy(est_hbm.at[ix2.at[sl]], rc_, sem).wait()
        pltpu.make_async_copy(est_hbm.at[ix3.at[sl]], rd_, sem).wait()

    def compute(b, accs):
        ra_, rb_, rc_, rd_, _ = bufs[b]

        def row2(rr, accs_in):
            outs = list(accs_in)
            for u in range(2):
                r = rr * 2 + u
                for h in range(2):
                    sl = pl.ds(h * 32, 32)
                    d = ((ra_[r, sl] - rb_[r, sl])
                         - (rc_[r, sl] - rd_[r, sl]))
                    lo, hi = plsc.unpack(d, format=plsc.PackFormat.INTERLEAVED)
                    j = 2 * h
                    outs[j] = outs[j] + lo * lo
                    outs[j + 1] = outs[j + 1] + hi * hi
            return tuple(outs)

        return lax.fori_loop(0, _CHUNK // 2, row2, accs)

    def superchunk(s, accs):
        base = pl.ds((chunk0 + s * _SCC) * _CHUNK, _SCC * _CHUNK)
        cps = [pltpu.async_copy(ig0_hbm.at[base], ix0, semi),
               pltpu.async_copy(ig1_hbm.at[base], ix1, semi),
               pltpu.async_copy(ie0_hbm.at[base], ix2, semi),
               pltpu.async_copy(ie1_hbm.at[base], ix3, semi)]
        for cp in cps:
            cp.wait()
        for b in range(_DEPTH):
            fire(b, b)

        def block(j, accs_in):
            for u in range(_DEPTH):
                wait(u)
                accs_in = compute(u, accs_in)
                fire(_DEPTH * j + u + _DEPTH, u)
            return accs_in

        accs = lax.fori_loop(0, _SCC // _DEPTH - 1, block, accs)
        for u in range(_DEPTH):
            wait(u)
            accs = compute(u, accs)
        return accs

    zero = jnp.zeros((_LANES,), jnp.float32)
    accs = lax.fori_loop(0, nsc, superchunk, (zero, zero, zero, zero))
    accv[...] = (accs[0] + accs[1]) + (accs[2] + accs[3])
    pltpu.sync_copy(accv, out_hbm.at[wid])


def kernel(gt_vertices, est_vertices, gt_connections, est_connections):
    B, N, C3 = gt_vertices.shape
    E = gt_connections.shape[0]
    row = C3 * B

    zpad = jnp.zeros((N, _ROWP - row), jnp.bfloat16)
    gtT = jnp.concatenate(
        [jnp.transpose(gt_vertices, (1, 2, 0)).reshape(N, row)
         .astype(jnp.bfloat16), zpad], axis=1)
    estT = jnp.concatenate(
        [jnp.transpose(est_vertices, (1, 2, 0)).reshape(N, row)
         .astype(jnp.bfloat16), zpad], axis=1)
    conn_g = gt_connections.astype(jnp.int32)
    conn_e = est_connections.astype(jnp.int32)
    z = jnp.zeros((_E_PAD - E,), jnp.int32)
    ig0 = jnp.concatenate([conn_g[:, 0], z])
    ig1 = jnp.concatenate([conn_g[:, 1], z])
    ie0 = jnp.concatenate([conn_e[:, 0], z])
    ie1 = jnp.concatenate([conn_e[:, 1], z])

    idxbuf = pltpu.VMEM((_SCC * _CHUNK,), jnp.int32)
    rowbuf = pltpu.VMEM((_CHUNK, _ROWP), jnp.bfloat16)
    run = pl.kernel(
        _sc_body,
        mesh=plsc.VectorSubcoreMesh(core_axis_name="c", subcore_axis_name="s"),
        compiler_params=pltpu.CompilerParams(use_tc_tiling_on_sc=False,
                                             needs_layout_passes=False),
        out_type=jax.ShapeDtypeStruct((_NW, _LANES), jnp.float32),
        scratch_types=(
            [idxbuf] * 4
            + [rowbuf] * (4 * _DEPTH)
            + [pltpu.VMEM((_LANES,), jnp.float32)]
            + [pltpu.SemaphoreType.DMA] * (1 + _DEPTH)
        ),
    )
    partials = run(gtT, estT, ig0, ig1, ie0, ie1)
    return jnp.sum(partials) / (B + 1e-08)


# speedup vs baseline: 3.0011x; 1.0457x over previous
import jax
import jax.numpy as jnp
from jax import lax
from jax.experimental import pallas as pl
from jax.experimental.pallas import tpu as pltpu
from jax.experimental.pallas import tpu_sc as plsc

_LANES = 16
_NC = 2
_NS = 16
_NW = _NC * _NS
_CHUNK = 128
_E_PAD = 819200
_NCHUNKS = _E_PAD // _CHUNK
_ROWP = 64
_SCC = 40
_DEPTH = 4
_CHUNKS_C0 = 360
_CHUNKS_C1 = 40


def _sc_body(gt_hbm, est_hbm, ig0_hbm, ig1_hbm, ie0_hbm, ie1_hbm, out_hbm,
             ix0, ix1, ix2, ix3,
             ra0, rb0, rc0, rd0, ra1, rb1, rc1, rd1,
             ra2, rb2, rc2, rd2, ra3, rb3, rc3, rd3,
             accv, semi, sem0, sem1, sem2, sem3):
    cid = lax.axis_index("c")
    sid = lax.axis_index("s")
    wid = sid * _NC + cid
    chunk0 = jnp.where(cid == 0, sid * _CHUNKS_C0,
                       _NS * _CHUNKS_C0 + sid * _CHUNKS_C1)
    nsc = jnp.where(cid == 0, _CHUNKS_C0 // _SCC, _CHUNKS_C1 // _SCC)

    bufs = ((ra0, rb0, rc0, rd0, sem0), (ra1, rb1, rc1, rd1, sem1),
            (ra2, rb2, rc2, rd2, sem2), (ra3, rb3, rc3, rd3, sem3))

    def fire(c, b):
        ra_, rb_, rc_, rd_, sem = bufs[b]
        sl = pl.ds(c * _CHUNK, _CHUNK)
        pltpu.async_copy(gt_hbm.at[ix0.at[sl]], ra_, sem)
        pltpu.async_copy(gt_hbm.at[ix1.at[sl]], rb_, sem)
        pltpu.async_copy(est_hbm.at[ix2.at[sl]], rc_, sem)
        pltpu.async_copy(est_hbm.at[ix3.at[sl]], rd_, sem)

    def wait(b):
        ra_, rb_, rc_, rd_, sem = bufs[b]
        sl = pl.ds(0, _CHUNK)
        pltpu.make_async_copy(gt_hbm.at[ix0.at[sl]], ra_, sem).wait()
        pltpu.make_async_copy(gt_hbm.at[ix1.at[sl]], rb_, sem).wait()
        pltpu.make_async_copy(est_hbm.at[ix2.at[sl]], rc_, sem).wait()
        pltpu.make_async_copy(est_hbm.at[ix3.at[sl]], rd_, sem).wait()

    def compute(b, accs):
        ra_, rb_, rc_, rd_, _ = bufs[b]

        def row2(rr, accs_in):
            outs = list(accs_in)
            for u in range(2):
                r = rr * 2 + u
                for h in range(2):
                    sl = pl.ds(h * 32, 32)
                    d = ((ra_[r, sl] - rb_[r, sl])
                         - (rc_[r, sl] - rd_[r, sl]))
                    lo, hi = plsc.unpack(d, format=plsc.PackFormat.INTERLEAVED)
                    j = 2 * h
                    outs[j] = outs[j] + lo * lo
                    outs[j + 1] = outs[j + 1] + hi * hi
            return tuple(outs)

        return lax.fori_loop(0, _CHUNK // 2, row2, accs)

    def superchunk(s, accs):
        base = pl.ds((chunk0 + s * _SCC) * _CHUNK, _SCC * _CHUNK)
        cps = [pltpu.async_copy(ig0_hbm.at[base], ix0, semi),
               pltpu.async_copy(ig1_hbm.at[base], ix1, semi),
               pltpu.async_copy(ie0_hbm.at[base], ix2, semi),
               pltpu.async_copy(ie1_hbm.at[base], ix3, semi)]
        for cp in cps:
            cp.wait()
        for b in range(_DEPTH):
            fire(b, b)

        def block(j, accs_in):
            for u in range(_DEPTH):
                wait(u)
                accs_in = compute(u, accs_in)
                fire(_DEPTH * j + u + _DEPTH, u)
            return accs_in

        accs = lax.fori_loop(0, _SCC // _DEPTH - 1, block, accs)
        for u in range(_DEPTH):
            wait(u)
            accs = compute(u, accs)
        return accs

    zero = jnp.zeros((_LANES,), jnp.float32)
    accs = lax.fori_loop(0, nsc, superchunk, (zero, zero, zero, zero))
    accv[...] = (accs[0] + accs[1]) + (accs[2] + accs[3])
    pltpu.sync_copy(accv, out_hbm.at[wid])


def kernel(gt_vertices, est_vertices, gt_connections, est_connections):
    B, N, C3 = gt_vertices.shape
    E = gt_connections.shape[0]
    row = C3 * B

    zpad = jnp.zeros((N, _ROWP - row), jnp.bfloat16)
    gtT = jnp.concatenate(
        [jnp.transpose(gt_vertices, (1, 2, 0)).reshape(N, row)
         .astype(jnp.bfloat16), zpad], axis=1)
    estT = jnp.concatenate(
        [jnp.transpose(est_vertices, (1, 2, 0)).reshape(N, row)
         .astype(jnp.bfloat16), zpad], axis=1)
    conn_g = gt_connections.astype(jnp.int32)
    conn_e = est_connections.astype(jnp.int32)
    z = jnp.zeros((_E_PAD - E,), jnp.int32)
    ig0 = jnp.concatenate([conn_g[:, 0], z])
    ig1 = jnp.concatenate([conn_g[:, 1], z])
    ie0 = jnp.concatenate([conn_e[:, 0], z])
    ie1 = jnp.concatenate([conn_e[:, 1], z])

    idxbuf = pltpu.VMEM((_SCC * _CHUNK,), jnp.int32)
    rowbuf = pltpu.VMEM((_CHUNK, _ROWP), jnp.bfloat16)
    run = pl.kernel(
        _sc_body,
        mesh=plsc.VectorSubcoreMesh(core_axis_name="c", subcore_axis_name="s"),
        compiler_params=pltpu.CompilerParams(use_tc_tiling_on_sc=False,
                                             needs_layout_passes=False),
        out_type=jax.ShapeDtypeStruct((_NW, _LANES), jnp.float32),
        scratch_types=(
            [idxbuf] * 4
            + [rowbuf] * (4 * _DEPTH)
            + [pltpu.VMEM((_LANES,), jnp.float32)]
            + [pltpu.SemaphoreType.DMA] * (1 + _DEPTH)
        ),
    )
    partials = run(gtT, estT, ig0, ig1, ie0, ie1)
    return jnp.sum(partials) / (B + 1e-08)
